# Initial kernel scaffold; baseline (speedup 1.0000x reference)
import jax
import jax.numpy as jnp
from jax.experimental import pallas as pl


def kernel(x, edge_index, Wenc, benc, Wc, bc, gamma, beta):
    raise NotImplementedError("write your pallas kernel here")


# speedup vs baseline: 18.0738x; 18.0738x over previous
import functools

import jax
import jax.numpy as jnp
from jax import lax
from jax.experimental import pallas as pl
from jax.experimental.pallas import tpu as pltpu
from jax.experimental.pallas import tpu_sc as plsc

N = 10000
E = 320000
D = 128
L = 3
BN_EPS = 1e-5

NC = 2
NS = 16
NW = NC * NS
K = 128
NCH = 79
EPAD = NW * NCH * K
NPAD = 10112
RPT = NPAD // NS
RB = 2528
GRID = NPAD // RB
XB = 2000
XGRID = N // XB


@functools.cache
def _mesh():
    return plsc.VectorSubcoreMesh(
        core_axis_name="c", subcore_axis_name="s", num_cores=NC, num_subcores=NS
    )


def _deg_body(dst_hbm, z16_hbm, ones_hbm, out_hbm, dst_v, ones_v, acc, gsem):
    del gsem
    c = lax.axis_index("c")
    s = lax.axis_index("s")
    w = c * NS + s
    pltpu.sync_copy(ones_hbm, ones_v)
    pltpu.sync_copy(z16_hbm.at[pl.ds(s * RPT, RPT)], acc.at[pl.ds(s * RPT, RPT)])
    pltpu.sync_copy(dst_hbm.at[w], dst_v)
    plsc.subcore_barrier()

    def chunk(j, _):
        pltpu.sync_copy(ones_v, acc.at[dst_v.at[j]], add=True)
        return 0

    lax.fori_loop(0, NCH, chunk, 0)
    plsc.subcore_barrier()
    pltpu.sync_copy(
        acc.at[pl.ds(s * RPT, RPT)], out_hbm.at[pl.ds(c * NPAD + s * RPT, RPT)]
    )


@functools.cache
def _deg_call():
    return pl.kernel(
        _deg_body,
        out_type=jax.ShapeDtypeStruct((NC * NPAD, D), jnp.float32),
        mesh=_mesh(),
        scratch_types=[
            pltpu.VMEM((NCH, K), jnp.int32),
            pltpu.VMEM((K, D), jnp.float32),
            pltpu.VMEM_SHARED((NPAD, D), jnp.float32),
            pltpu.SemaphoreType.DMA,
        ],
    )


def _scatter_body(src_hbm, dst_hbm, g_hbm, zd_hbm, out_hbm, src_v, dst_v, buf0, acc, gsem):
    c = lax.axis_index("c")
    s = lax.axis_index("s")
    w = c * NS + s
    pltpu.sync_copy(zd_hbm.at[pl.ds(s * RPT, RPT)], acc.at[pl.ds(s * RPT, RPT)])
    pltpu.sync_copy(src_hbm.at[w], src_v)
    pltpu.sync_copy(dst_hbm.at[w], dst_v)
    plsc.subcore_barrier()

    def chunk(j, _):
        pltpu.async_copy(g_hbm.at[src_v.at[j]], buf0, gsem).wait()
        pltpu.sync_copy(buf0, acc.at[dst_v.at[j]], add=True)
        return 0

    lax.fori_loop(0, NCH, chunk, 0)
    plsc.subcore_barrier()
    pltpu.sync_copy(
        acc.at[pl.ds(s * RPT, RPT)], out_hbm.at[pl.ds(c * NPAD + s * RPT, RPT)]
    )


@functools.cache
def _scatter_call():
    return pl.kernel(
        _scatter_body,
        out_type=jax.ShapeDtypeStruct((NC * NPAD, D), jnp.float32),
        mesh=_mesh(),
        scratch_types=[
            pltpu.VMEM((NCH, K), jnp.int32),
            pltpu.VMEM((NCH, K), jnp.int32),
            pltpu.VMEM((K, D), jnp.float32),
            pltpu.VMEM_SHARED((NPAD, D), jnp.float32),
            pltpu.SemaphoreType.DMA,
        ],
    )


def _enc_body(x_ref, wenc_ref, benc_ref, wc_ref, out_ref):
    h = jnp.dot(x_ref[...], wenc_ref[...], preferred_element_type=jnp.float32)
    h = h + benc_ref[...]
    out_ref[...] = jnp.dot(h, wc_ref[0], preferred_element_type=jnp.float32)


_enc_call = pl.pallas_call(
    _enc_body,
    grid=(XGRID,),
    in_specs=[
        pl.BlockSpec((XB, D), lambda i: (i, 0)),
        pl.BlockSpec((D, D), lambda i: (0, 0)),
        pl.BlockSpec((1, D), lambda i: (0, 0)),
        pl.BlockSpec((1, D, D), lambda i: (0, 0, 0)),
    ],
    out_specs=pl.BlockSpec((XB, D), lambda i: (i, 0)),
    out_shape=jax.ShapeDtypeStruct((NPAD, D), jnp.float32),
)


def _scale_body(deg_ref, hw_ref, g_ref, dinv_ref):
    i = pl.program_id(0)
    d2 = deg_ref[...]
    degs = d2[0, :, 0:1] + d2[1, :, 0:1] + 1.0
    dcol = lax.rsqrt(jnp.maximum(degs, 1.0))
    rows = i * RB + lax.broadcasted_iota(jnp.int32, (RB, 1), 0)
    mask = rows < N
    g_ref[...] = jnp.where(mask, dcol * hw_ref[...], 0.0)
    dinv_ref[...] = jnp.where(mask, dcol, 0.0)


_scale_call = pl.pallas_call(
    _scale_body,
    grid=(GRID,),
    in_specs=[
        pl.BlockSpec((NC, RB, D), lambda i: (0, i, 0)),
        pl.BlockSpec((RB, D), lambda i: (i, 0)),
    ],
    out_specs=[
        pl.BlockSpec((RB, D), lambda i: (i, 0)),
        pl.BlockSpec((RB, 1), lambda i: (i, 0)),
    ],
    out_shape=[
        jax.ShapeDtypeStruct((NPAD, D), jnp.float32),
        jax.ShapeDtypeStruct((NPAD, 1), jnp.float32),
    ],
)


def _update_body(l, s_ref, g_ref, dinv_ref, wc_ref, bc_ref, gam_ref, bet_ref, out_ref):
    i = pl.program_id(0)
    s2 = s_ref[...]
    agg = s2[0] + s2[1] + g_ref[...]
    dcol = dinv_ref[...]
    out = dcol * agg + bc_ref[0]
    inv = (1.0 + BN_EPS) ** -0.5
    h = out * (gam_ref[0] * inv) + bet_ref[0]
    if l < L - 1:
        h = jnp.maximum(h, 0.0)
        hw = jnp.dot(h, wc_ref[0], preferred_element_type=jnp.float32)
        rows = i * RB + lax.broadcasted_iota(jnp.int32, (RB, 1), 0)
        out_ref[...] = jnp.where(rows < N, dcol * hw, 0.0)
    else:
        out_ref[...] = h


def _make_update(l):
    wnext = min(l + 1, L - 1)
    out_rows = NPAD if l < L - 1 else N
    return pl.pallas_call(
        functools.partial(_update_body, l),
        grid=(GRID,),
        in_specs=[
            pl.BlockSpec((NC, RB, D), lambda i: (0, i, 0)),
            pl.BlockSpec((RB, D), lambda i: (i, 0)),
            pl.BlockSpec((RB, 1), lambda i: (i, 0)),
            pl.BlockSpec((1, D, D), lambda i, w=wnext: (w, 0, 0)),
            pl.BlockSpec((1, 1, D), lambda i, w=l: (w, 0, 0)),
            pl.BlockSpec((1, 1, D), lambda i, w=l: (w, 0, 0)),
            pl.BlockSpec((1, 1, D), lambda i, w=l: (w, 0, 0)),
        ],
        out_specs=pl.BlockSpec((RB, D), lambda i: (i, 0)),
        out_shape=jax.ShapeDtypeStruct((out_rows, D), jnp.float32),
    )


_update_calls = [_make_update(l) for l in range(L)]


def kernel(x, edge_index, Wenc, benc, Wc, bc, gamma, beta):
    src = edge_index[0]
    dst = edge_index[1]
    pad = EPAD - E
    pad_idx = (jnp.arange(pad, dtype=jnp.int32) % (NPAD - N)) + N
    srcp = jnp.concatenate([src, pad_idx]).reshape(NW, NCH, K)
    dstp = jnp.concatenate([dst, pad_idx]).reshape(NW, NCH, K)
    zd = jnp.zeros((NPAD, D), jnp.float32)

    onesd = jnp.ones((K, D), jnp.float32)
    deg = _deg_call()(dstp, zd, onesd).reshape(NC, NPAD, D)
    h0w = _enc_call(x.astype(jnp.float32), Wenc, benc.reshape(1, D), Wc)
    g, dinv = _scale_call(deg, h0w)
    bc3 = bc.reshape(L, 1, D)
    gam3 = gamma.reshape(L, 1, D)
    bet3 = beta.reshape(L, 1, D)
    for l in range(L):
        s_part = _scatter_call()(srcp, dstp, g, zd).reshape(NC, NPAD, D)
        g = _update_calls[l](s_part, g, dinv, Wc, bc3, gam3, bet3)
    return g


# speedup vs baseline: 18.3413x; 1.0148x over previous
import functools

import jax
import jax.numpy as jnp
from jax import lax
from jax.experimental import pallas as pl
from jax.experimental.pallas import tpu as pltpu
from jax.experimental.pallas import tpu_sc as plsc

N = 10000
E = 320000
D = 128
L = 3
BN_EPS = 1e-5

NC = 2
NS = 16
NW = NC * NS
K = 128
NCH = 79
EPAD = NW * NCH * K
NPAD = 10112
RPT = NPAD // NS
RB = 2528
GRID = NPAD // RB
XB = 2000
XGRID = N // XB


@functools.cache
def _mesh():
    return plsc.VectorSubcoreMesh(
        core_axis_name="c", subcore_axis_name="s", num_cores=NC, num_subcores=NS
    )


def _deg_body(dst_hbm, z16_hbm, ones_hbm, out_hbm, dst_v, ones_v, acc, gsem):
    c = lax.axis_index("c")
    s = lax.axis_index("s")
    w = c * NS + s
    d0 = pltpu.async_copy(ones_hbm, ones_v, gsem)
    d1 = pltpu.async_copy(
        z16_hbm.at[pl.ds(s * RPT, RPT)], acc.at[pl.ds(s * RPT, RPT)], gsem
    )
    d2 = pltpu.async_copy(dst_hbm.at[w], dst_v, gsem)
    d0.wait()
    d1.wait()
    d2.wait()
    plsc.subcore_barrier()

    def chunk(j, _):
        pltpu.sync_copy(ones_v, acc.at[dst_v.at[j]], add=True)
        return 0

    lax.fori_loop(0, NCH, chunk, 0)
    plsc.subcore_barrier()
    pltpu.sync_copy(
        acc.at[pl.ds(s * RPT, RPT)], out_hbm.at[pl.ds(c * NPAD + s * RPT, RPT)]
    )


@functools.cache
def _deg_call():
    return pl.kernel(
        _deg_body,
        out_type=jax.ShapeDtypeStruct((NC * NPAD, D), jnp.float32),
        mesh=_mesh(),
        scratch_types=[
            pltpu.VMEM((NCH, K), jnp.int32),
            pltpu.VMEM((K, D), jnp.float32),
            pltpu.VMEM_SHARED((NPAD, D), jnp.float32),
            pltpu.SemaphoreType.DMA,
        ],
    )


def _scatter_body(src_hbm, dst_hbm, g_hbm, zd_hbm, out_hbm, src_v, dst_v, buf0, acc, gsem):
    c = lax.axis_index("c")
    s = lax.axis_index("s")
    w = c * NS + s
    d0 = pltpu.async_copy(
        zd_hbm.at[pl.ds(s * RPT, RPT)], acc.at[pl.ds(s * RPT, RPT)], gsem
    )
    d1 = pltpu.async_copy(src_hbm.at[w], src_v, gsem)
    d2 = pltpu.async_copy(dst_hbm.at[w], dst_v, gsem)
    d0.wait()
    d1.wait()
    d2.wait()
    plsc.subcore_barrier()

    def chunk(j, _):
        pltpu.async_copy(g_hbm.at[src_v.at[j]], buf0, gsem).wait()
        pltpu.sync_copy(buf0, acc.at[dst_v.at[j]], add=True)
        return 0

    lax.fori_loop(0, NCH, chunk, 0)
    plsc.subcore_barrier()
    pltpu.sync_copy(
        acc.at[pl.ds(s * RPT, RPT)], out_hbm.at[pl.ds(c * NPAD + s * RPT, RPT)]
    )


@functools.cache
def _scatter_call():
    return pl.kernel(
        _scatter_body,
        out_type=jax.ShapeDtypeStruct((NC * NPAD, D), jnp.float32),
        mesh=_mesh(),
        scratch_types=[
            pltpu.VMEM((NCH, K), jnp.int32),
            pltpu.VMEM((NCH, K), jnp.int32),
            pltpu.VMEM((K, D), jnp.float32),
            pltpu.VMEM_SHARED((NPAD, D), jnp.float32),
            pltpu.SemaphoreType.DMA,
        ],
    )


def _enc_body(x_ref, wenc_ref, benc_ref, wc_ref, out_ref):
    h = jnp.dot(x_ref[...], wenc_ref[...], preferred_element_type=jnp.float32)
    h = h + benc_ref[...]
    out_ref[...] = jnp.dot(h, wc_ref[0], preferred_element_type=jnp.float32)


_enc_call = pl.pallas_call(
    _enc_body,
    grid=(XGRID,),
    in_specs=[
        pl.BlockSpec((XB, D), lambda i: (i, 0)),
        pl.BlockSpec((D, D), lambda i: (0, 0)),
        pl.BlockSpec((1, D), lambda i: (0, 0)),
        pl.BlockSpec((1, D, D), lambda i: (0, 0, 0)),
    ],
    out_specs=pl.BlockSpec((XB, D), lambda i: (i, 0)),
    out_shape=jax.ShapeDtypeStruct((NPAD, D), jnp.float32),
)


def _scale_body(deg_ref, hw_ref, g_ref, dinv_ref):
    i = pl.program_id(0)
    d2 = deg_ref[...]
    degs = d2[0, :, 0:1] + d2[1, :, 0:1] + 1.0
    dcol = lax.rsqrt(jnp.maximum(degs, 1.0))
    rows = i * RB + lax.broadcasted_iota(jnp.int32, (RB, 1), 0)
    mask = rows < N
    g_ref[...] = jnp.where(mask, dcol * hw_ref[...], 0.0)
    dinv_ref[...] = jnp.where(mask, dcol, 0.0)


_scale_call = pl.pallas_call(
    _scale_body,
    grid=(GRID,),
    in_specs=[
        pl.BlockSpec((NC, RB, D), lambda i: (0, i, 0)),
        pl.BlockSpec((RB, D), lambda i: (i, 0)),
    ],
    out_specs=[
        pl.BlockSpec((RB, D), lambda i: (i, 0)),
        pl.BlockSpec((RB, 1), lambda i: (i, 0)),
    ],
    out_shape=[
        jax.ShapeDtypeStruct((NPAD, D), jnp.float32),
        jax.ShapeDtypeStruct((NPAD, 1), jnp.float32),
    ],
)


def _update_body(l, s_ref, g_ref, dinv_ref, wc_ref, bc_ref, gam_ref, bet_ref, out_ref):
    i = pl.program_id(0)
    s2 = s_ref[...]
    agg = s2[0] + s2[1] + g_ref[...]
    dcol = dinv_ref[...]
    out = dcol * agg + bc_ref[0]
    inv = (1.0 + BN_EPS) ** -0.5
    h = out * (gam_ref[0] * inv) + bet_ref[0]
    if l < L - 1:
        h = jnp.maximum(h, 0.0)
        hw = jnp.dot(h, wc_ref[0], preferred_element_type=jnp.float32)
        rows = i * RB + lax.broadcasted_iota(jnp.int32, (RB, 1), 0)
        out_ref[...] = jnp.where(rows < N, dcol * hw, 0.0)
    else:
        out_ref[...] = h


def _make_update(l):
    wnext = min(l + 1, L - 1)
    out_rows = NPAD if l < L - 1 else N
    return pl.pallas_call(
        functools.partial(_update_body, l),
        grid=(GRID,),
        in_specs=[
            pl.BlockSpec((NC, RB, D), lambda i: (0, i, 0)),
            pl.BlockSpec((RB, D), lambda i: (i, 0)),
            pl.BlockSpec((RB, 1), lambda i: (i, 0)),
            pl.BlockSpec((1, D, D), lambda i, w=wnext: (w, 0, 0)),
            pl.BlockSpec((1, 1, D), lambda i, w=l: (w, 0, 0)),
            pl.BlockSpec((1, 1, D), lambda i, w=l: (w, 0, 0)),
            pl.BlockSpec((1, 1, D), lambda i, w=l: (w, 0, 0)),
        ],
        out_specs=pl.BlockSpec((RB, D), lambda i: (i, 0)),
        out_shape=jax.ShapeDtypeStruct((out_rows, D), jnp.float32),
    )


_update_calls = [_make_update(l) for l in range(L)]


def kernel(x, edge_index, Wenc, benc, Wc, bc, gamma, beta):
    src = edge_index[0]
    dst = edge_index[1]
    pad = EPAD - E
    pad_idx = (jnp.arange(pad, dtype=jnp.int32) % (NPAD - N)) + N
    srcp = jnp.concatenate([src, pad_idx]).reshape(NW, NCH, K)
    dstp = jnp.concatenate([dst, pad_idx]).reshape(NW, NCH, K)
    zd = jnp.zeros((NPAD, D), jnp.float32)

    onesd = jnp.ones((K, D), jnp.float32)
    deg = _deg_call()(dstp, zd, onesd).reshape(NC, NPAD, D)
    h0w = _enc_call(x.astype(jnp.float32), Wenc, benc.reshape(1, D), Wc)
    g, dinv = _scale_call(deg, h0w)
    bc3 = bc.reshape(L, 1, D)
    gam3 = gamma.reshape(L, 1, D)
    bet3 = beta.reshape(L, 1, D)
    for l in range(L):
        s_part = _scatter_call()(srcp, dstp, g, zd).reshape(NC, NPAD, D)
        g = _update_calls[l](s_part, g, dinv, Wc, bc3, gam3, bet3)
    return g


# speedup vs baseline: 18.4078x; 1.0036x over previous
import functools

import jax
import jax.numpy as jnp
from jax import lax
from jax.experimental import pallas as pl
from jax.experimental.pallas import tpu as pltpu
from jax.experimental.pallas import tpu_sc as plsc

N = 10000
E = 320000
D = 128
L = 3
BN_EPS = 1e-5

NC = 2
NS = 16
NW = NC * NS
K = 128
NCH = 79
EPAD = NW * NCH * K
NPAD = 10112
RPT = NPAD // NS
RB = 2528
GRID = NPAD // RB
XB = 2000
XGRID = N // XB


@functools.cache
def _mesh():
    return plsc.VectorSubcoreMesh(
        core_axis_name="c", subcore_axis_name="s", num_cores=NC, num_subcores=NS
    )


def _deg_body(dst_hbm, z16_hbm, ones_hbm, out_hbm, dst_v, ones_v, acc, gsem):
    c = lax.axis_index("c")
    s = lax.axis_index("s")
    w = c * NS + s
    d0 = pltpu.async_copy(ones_hbm, ones_v, gsem)
    d1 = pltpu.async_copy(
        z16_hbm.at[pl.ds(s * RPT, RPT)], acc.at[pl.ds(s * RPT, RPT)], gsem
    )
    d2 = pltpu.async_copy(dst_hbm.at[w], dst_v, gsem)
    d0.wait()
    d1.wait()
    d2.wait()
    plsc.subcore_barrier()

    def chunk(j, _):
        pltpu.sync_copy(ones_v, acc.at[dst_v.at[j]], add=True)
        return 0

    lax.fori_loop(0, NCH, chunk, 0)
    plsc.subcore_barrier()
    pltpu.sync_copy(
        acc.at[pl.ds(s * RPT, RPT)], out_hbm.at[pl.ds(c * NPAD + s * RPT, RPT)]
    )


@functools.cache
def _deg_call():
    return pl.kernel(
        _deg_body,
        out_type=jax.ShapeDtypeStruct((NC * NPAD, D), jnp.float32),
        mesh=_mesh(),
        scratch_types=[
            pltpu.VMEM((NCH, K), jnp.int32),
            pltpu.VMEM((K, D), jnp.float32),
            pltpu.VMEM_SHARED((NPAD, D), jnp.float32),
            pltpu.SemaphoreType.DMA,
        ],
    )


def _scatter_body(src_hbm, dst_hbm, g_hbm, zd_hbm, out_hbm, src_v, dst_v, buf0, acc, gsem, isem):
    c = lax.axis_index("c")
    s = lax.axis_index("s")
    w = c * NS + s
    d0 = pltpu.async_copy(
        zd_hbm.at[pl.ds(s * RPT, RPT)], acc.at[pl.ds(s * RPT, RPT)], gsem
    )
    d1 = pltpu.async_copy(src_hbm.at[w], src_v, isem)
    d2 = pltpu.async_copy(dst_hbm.at[w], dst_v, gsem)
    d1.wait()
    dg = pltpu.async_copy(g_hbm.at[src_v.at[0]], buf0, isem)
    d0.wait()
    d2.wait()
    plsc.subcore_barrier()
    dg.wait()
    pltpu.sync_copy(buf0, acc.at[dst_v.at[0]], add=True)

    def chunk(j, _):
        pltpu.async_copy(g_hbm.at[src_v.at[j]], buf0, gsem).wait()
        pltpu.sync_copy(buf0, acc.at[dst_v.at[j]], add=True)
        return 0

    lax.fori_loop(1, NCH, chunk, 0)
    plsc.subcore_barrier()
    pltpu.sync_copy(
        acc.at[pl.ds(s * RPT, RPT)], out_hbm.at[pl.ds(c * NPAD + s * RPT, RPT)]
    )


@functools.cache
def _scatter_call():
    return pl.kernel(
        _scatter_body,
        out_type=jax.ShapeDtypeStruct((NC * NPAD, D), jnp.float32),
        mesh=_mesh(),
        scratch_types=[
            pltpu.VMEM((NCH, K), jnp.int32),
            pltpu.VMEM((NCH, K), jnp.int32),
            pltpu.VMEM((K, D), jnp.float32),
            pltpu.VMEM_SHARED((NPAD, D), jnp.float32),
            pltpu.SemaphoreType.DMA,
            pltpu.SemaphoreType.DMA,
        ],
    )


def _enc_body(x_ref, wenc_ref, benc_ref, wc_ref, out_ref):
    h = jnp.dot(x_ref[...], wenc_ref[...], preferred_element_type=jnp.float32)
    h = h + benc_ref[...]
    out_ref[...] = jnp.dot(h, wc_ref[0], preferred_element_type=jnp.float32)


_enc_call = pl.pallas_call(
    _enc_body,
    grid=(XGRID,),
    in_specs=[
        pl.BlockSpec((XB, D), lambda i: (i, 0)),
        pl.BlockSpec((D, D), lambda i: (0, 0)),
        pl.BlockSpec((1, D), lambda i: (0, 0)),
        pl.BlockSpec((1, D, D), lambda i: (0, 0, 0)),
    ],
    out_specs=pl.BlockSpec((XB, D), lambda i: (i, 0)),
    out_shape=jax.ShapeDtypeStruct((NPAD, D), jnp.float32),
)


def _scale_body(deg_ref, hw_ref, g_ref, dinv_ref):
    i = pl.program_id(0)
    d2 = deg_ref[...]
    degs = d2[0, :, 0:1] + d2[1, :, 0:1] + 1.0
    dcol = lax.rsqrt(jnp.maximum(degs, 1.0))
    rows = i * RB + lax.broadcasted_iota(jnp.int32, (RB, 1), 0)
    mask = rows < N
    g_ref[...] = jnp.where(mask, dcol * hw_ref[...], 0.0)
    dinv_ref[...] = jnp.where(mask, dcol, 0.0)


_scale_call = pl.pallas_call(
    _scale_body,
    grid=(GRID,),
    in_specs=[
        pl.BlockSpec((NC, RB, D), lambda i: (0, i, 0)),
        pl.BlockSpec((RB, D), lambda i: (i, 0)),
    ],
    out_specs=[
        pl.BlockSpec((RB, D), lambda i: (i, 0)),
        pl.BlockSpec((RB, 1), lambda i: (i, 0)),
    ],
    out_shape=[
        jax.ShapeDtypeStruct((NPAD, D), jnp.float32),
        jax.ShapeDtypeStruct((NPAD, 1), jnp.float32),
    ],
)


def _update_body(l, s_ref, g_ref, dinv_ref, wc_ref, bc_ref, gam_ref, bet_ref, out_ref):
    i = pl.program_id(0)
    s2 = s_ref[...]
    agg = s2[0] + s2[1] + g_ref[...]
    dcol = dinv_ref[...]
    out = dcol * agg + bc_ref[0]
    inv = (1.0 + BN_EPS) ** -0.5
    h = out * (gam_ref[0] * inv) + bet_ref[0]
    if l < L - 1:
        h = jnp.maximum(h, 0.0)
        hw = jnp.dot(h, wc_ref[0], preferred_element_type=jnp.float32)
        rows = i * RB + lax.broadcasted_iota(jnp.int32, (RB, 1), 0)
        out_ref[...] = jnp.where(rows < N, dcol * hw, 0.0)
    else:
        out_ref[...] = h


def _make_update(l):
    wnext = min(l + 1, L - 1)
    out_rows = NPAD if l < L - 1 else N
    return pl.pallas_call(
        functools.partial(_update_body, l),
        grid=(GRID,),
        in_specs=[
            pl.BlockSpec((NC, RB, D), lambda i: (0, i, 0)),
            pl.BlockSpec((RB, D), lambda i: (i, 0)),
            pl.BlockSpec((RB, 1), lambda i: (i, 0)),
            pl.BlockSpec((1, D, D), lambda i, w=wnext: (w, 0, 0)),
            pl.BlockSpec((1, 1, D), lambda i, w=l: (w, 0, 0)),
            pl.BlockSpec((1, 1, D), lambda i, w=l: (w, 0, 0)),
            pl.BlockSpec((1, 1, D), lambda i, w=l: (w, 0, 0)),
        ],
        out_specs=pl.BlockSpec((RB, D), lambda i: (i, 0)),
        out_shape=jax.ShapeDtypeStruct((out_rows, D), jnp.float32),
    )


_update_calls = [_make_update(l) for l in range(L)]


def kernel(x, edge_index, Wenc, benc, Wc, bc, gamma, beta):
    src = edge_index[0]
    dst = edge_index[1]
    pad = EPAD - E
    pad_idx = (jnp.arange(pad, dtype=jnp.int32) % (NPAD - N)) + N
    srcp = jnp.concatenate([src, pad_idx]).reshape(NW, NCH, K)
    dstp = jnp.concatenate([dst, pad_idx]).reshape(NW, NCH, K)
    zd = jnp.zeros((NPAD, D), jnp.float32)

    onesd = jnp.ones((K, D), jnp.float32)
    deg = _deg_call()(dstp, zd, onesd).reshape(NC, NPAD, D)
    h0w = _enc_call(x.astype(jnp.float32), Wenc, benc.reshape(1, D), Wc)
    g, dinv = _scale_call(deg, h0w)
    bc3 = bc.reshape(L, 1, D)
    gam3 = gamma.reshape(L, 1, D)
    bet3 = beta.reshape(L, 1, D)
    for l in range(L):
        s_part = _scatter_call()(srcp, dstp, g, zd).reshape(NC, NPAD, D)
        g = _update_calls[l](s_part, g, dinv, Wc, bc3, gam3, bet3)
    return g
